# baseline (speedup 1.0000x reference)
import functools

import jax
import jax.numpy as jnp
import numpy as np
from jax import lax
from jax.experimental import pallas as pl
from jax.experimental.pallas import tpu as pltpu
from jax.experimental.pallas import tpu_sc as plsc

B = 16384
V = 100000
D = 128

_NC = 2
_NS = 16
_NW = _NC * _NS
_BPW = B // _NW
_R = 128
_NCH = _BPW // _R
_TP = 137
_L = 16
_NDJ = D // _L

_MAGIC = np.int32(0x5F3759DF)

_GDN = lax.GatherDimensionNumbers(
    offset_dims=(), collapsed_slice_dims=(0,), start_index_map=(0,))


def _bcast(v, l):
    idx = jnp.full((_L, 1), l, jnp.int32)
    return lax.gather(v, idx, _GDN, slice_sizes=(1,),
                      mode=lax.GatherScatterMode.PROMISE_IN_BOUNDS)


def _rsqrt16(x):
    i = plsc.bitcast(x, jnp.int32)
    i = _MAGIC - lax.shift_right_logical(i, 1)
    y = plsc.bitcast(i, jnp.float32)
    hx = x * 0.5
    for _ in range(3):
        y = y * (1.5 - hx * y * y)
    return y


def _sc_fused(idx3, table, cx, cy, ng, w0, w1, gb, nb):
    mesh = plsc.VectorSubcoreMesh(core_axis_name="c", subcore_axis_name="s")

    @functools.partial(
        pl.kernel,
        out_type=jax.ShapeDtypeStruct((D, B), jnp.float32),
        mesh=mesh,
        scratch_types=[
            pltpu.VMEM((_NCH, _R), jnp.int32),
            pltpu.VMEM((2, _R, D), jnp.float32),
            pltpu.VMEM((2, D, _TP), jnp.float32),
            pltpu.VMEM((_L * 17,), jnp.float32),
            pltpu.VMEM((_BPW,), jnp.float32),
            pltpu.VMEM((_BPW,), jnp.float32),
            pltpu.VMEM((_BPW,), jnp.float32),
            pltpu.VMEM((D,), jnp.float32),
            pltpu.VMEM((D,), jnp.float32),
            pltpu.VMEM((D,), jnp.float32),
            pltpu.VMEM((D,), jnp.float32),
            pltpu.SemaphoreType.DMA,
            pltpu.SemaphoreType.DMA,
        ],
        compiler_params=pltpu.CompilerParams(needs_layout_passes=False),
    )
    def k(idx_hbm, table_hbm, cx_hbm, cy_hbm, ng_hbm, w0_hbm, w1_hbm,
          gb_hbm, nb_hbm, out_hbm,
          idx_v, rows_v, trans_v, tb_v, cx_v, cy_v, ng_v,
          w0_v, w1_v, gb_v, nb_v, gsem, osem):
        wid = lax.axis_index("s") * _NC + lax.axis_index("c")
        base = wid * _BPW

        pltpu.sync_copy(idx_hbm.at[wid], idx_v)
        pltpu.sync_copy(cx_hbm.at[pl.ds(base, _BPW)], cx_v)
        pltpu.sync_copy(cy_hbm.at[pl.ds(base, _BPW)], cy_v)
        pltpu.sync_copy(ng_hbm.at[pl.ds(base, _BPW)], ng_v)
        pltpu.sync_copy(w0_hbm, w0_v)
        pltpu.sync_copy(w1_hbm, w1_v)
        pltpu.sync_copy(gb_hbm, gb_v)
        pltpu.sync_copy(nb_hbm, nb_v)

        lane = lax.iota(jnp.int32, _L)
        lane17 = lane * 17
        dlanes = [lane + dj * _L for dj in range(_NDJ)]

        first = pltpu.async_copy(table_hbm.at[idx_v.at[0]], rows_v.at[0],
                                 gsem)
        pending = [first]
        out_pending = [None, None]

        for c in range(_NCH):
            cb = c % 2
            if c + 1 < _NCH:
                pending.append(
                    pltpu.async_copy(table_hbm.at[idx_v.at[c + 1]],
                                     rows_v.at[(c + 1) % 2], gsem))
            pending.pop(0).wait()
            if out_pending[cb] is not None:
                out_pending[cb].wait()
                out_pending[cb] = None

            def g_body(g, _):
                gof = g * _L
                w0b = [w0_v[pl.ds(dj * _L, _L)] for dj in range(_NDJ)]
                w1b = [w1_v[pl.ds(dj * _L, _L)] for dj in range(_NDJ)]
                gbb = [gb_v[pl.ds(dj * _L, _L)] for dj in range(_NDJ)]
                nbg = [nb_v[pl.ds(dj * _L, _L)] - gbb[dj]
                       for dj in range(_NDJ)]

                for l in range(_L):
                    r = gof + l
                    a0 = jnp.zeros((_L,), jnp.float32)
                    a1 = jnp.zeros((_L,), jnp.float32)
                    for dj in range(_NDJ):
                        v = rows_v[cb, r, pl.ds(dj * _L, _L)]
                        if dj % 2 == 0:
                            a0 = a0 + v * v
                        else:
                            a1 = a1 + v * v
                    plsc.store_scatter(tb_v, [lane17 + l], a0 + a1)
                n2 = tb_v[pl.ds(0, _L)]
                for i in range(1, _L):
                    n2 = n2 + tb_v[pl.ds(i * 17, _L)]
                rn = _rsqrt16(n2)

                ngb = ng_v[pl.ds(c * _R + gof, _L)]
                sc1 = 1.0 - ngb
                ab = cx_v[pl.ds(c * _R + gof, _L)] * sc1
                bb = cy_v[pl.ds(c * _R + gof, _L)] * sc1

                cbv = jnp.full((_L,), cb, jnp.int32)
                for l in range(_L):
                    r = gof + l
                    srn = _bcast(rn, l)
                    sa = _bcast(ab, l)
                    sb = _bcast(bb, l)
                    sng = _bcast(ngb, l)
                    col = jnp.full((_L,), r, jnp.int32)
                    for dj in range(_NDJ):
                        v = rows_v[cb, r, pl.ds(dj * _L, _L)]
                        res = (v * srn + sa * w0b[dj] + sb * w1b[dj]
                               + (gbb[dj] + sng * nbg[dj]))
                        plsc.store_scatter(trans_v,
                                           [cbv, dlanes[dj], col], res)
                return 0

            lax.fori_loop(0, _R // _L, g_body, 0)

            out_pending[cb] = pltpu.async_copy(
                trans_v.at[cb, :, pl.ds(0, _R)],
                out_hbm.at[:, pl.ds(base + c * _R, _R)],
                osem)

        for cp in out_pending:
            if cp is not None:
                cp.wait()

    return k(idx3, table, cx, cy, ng, w0, w1, gb, nb)


def kernel(nodes, coords, nogeo, table, geo_W, geo_B, nogeo_embed):
    idx3 = nodes.astype(jnp.int32).reshape(_NW, _NCH, _R)
    cxy = coords.T
    return _sc_fused(idx3, table, cxy[0], cxy[1], nogeo,
                     geo_W[0], geo_W[1], geo_B[0], nogeo_embed[0])


# speedup vs baseline: 1.9952x; 1.9952x over previous
import functools

import jax
import jax.numpy as jnp
from jax import lax
from jax.experimental import pallas as pl
from jax.experimental.pallas import tpu as pltpu
from jax.experimental.pallas import tpu_sc as plsc

B = 16384
V = 100000
D = 128

_NC = 2
_NS = 16
_NW = _NC * _NS
_BPW = B // _NW
_CHUNK = 128
_NCHUNK = _BPW // _CHUNK


def _sc_gather(table, idx):
    mesh = plsc.VectorSubcoreMesh(core_axis_name="c", subcore_axis_name="s")

    @functools.partial(
        pl.kernel,
        out_type=jax.ShapeDtypeStruct((B, D), jnp.float32),
        mesh=mesh,
        scratch_types=[
            pltpu.VMEM((_NCHUNK, _CHUNK), jnp.int32),
            pltpu.VMEM((_BPW, D), jnp.float32),
            pltpu.SemaphoreType.DMA,
        ],
    )
    def gather_kernel(idx_hbm, table_hbm, out_hbm, idx_v, rows_v, sem):
        wid = lax.axis_index("s") * _NC + lax.axis_index("c")
        base = wid * _BPW
        pltpu.sync_copy(idx_hbm.at[wid], idx_v)
        copies = []
        for c in range(_NCHUNK):
            copies.append(
                pltpu.async_copy(
                    table_hbm.at[idx_v.at[c]],
                    rows_v.at[pl.ds(c * _CHUNK, _CHUNK)],
                    sem,
                )
            )
        for cp in copies:
            cp.wait()
        pltpu.sync_copy(rows_v, out_hbm.at[pl.ds(base, _BPW)])

    return gather_kernel(idx.reshape(_NW, _NCHUNK, _CHUNK), table)


def _tc_dense_kernel(rows_ref, coords_ref, nogeo_ref, w_ref, b_ref, nb_ref,
                     out_ref):
    r = rows_ref[...]
    n2 = jnp.sum(r * r, axis=1, keepdims=True)
    inv = lax.rsqrt(n2)
    ng = nogeo_ref[...]
    scale = 1.0 - ng
    cx = coords_ref[:, 0:1] * scale
    cy = coords_ref[:, 1:2] * scale
    w0 = w_ref[0:1, :]
    w1 = w_ref[1:2, :]
    pos = cx * w0 + cy * w1 + b_ref[...] + ng * (nb_ref[...] - b_ref[...])
    res = r * inv + pos
    out_ref[...] = res.T


def _tc_dense(rows, coords, nogeo2d, geo_W, geo_B, nogeo_embed):
    BB = 1024
    grid = B // BB
    return pl.pallas_call(
        _tc_dense_kernel,
        grid=(grid,),
        in_specs=[
            pl.BlockSpec((BB, D), lambda i: (i, 0)),
            pl.BlockSpec((BB, 2), lambda i: (i, 0)),
            pl.BlockSpec((BB, 1), lambda i: (i, 0)),
            pl.BlockSpec((2, D), lambda i: (0, 0)),
            pl.BlockSpec((1, D), lambda i: (0, 0)),
            pl.BlockSpec((1, D), lambda i: (0, 0)),
        ],
        out_specs=pl.BlockSpec((D, BB), lambda i: (0, i)),
        out_shape=jax.ShapeDtypeStruct((D, B), jnp.float32),
    )(rows, coords, nogeo2d, geo_W, geo_B, nogeo_embed)


def kernel(nodes, coords, nogeo, table, geo_W, geo_B, nogeo_embed):
    idx = nodes.astype(jnp.int32)
    rows = _sc_gather(table, idx)
    return _tc_dense(rows, coords, nogeo.reshape(B, 1), geo_W, geo_B,
                     nogeo_embed)


# speedup vs baseline: 2.1533x; 1.0792x over previous
import functools

import jax
import jax.numpy as jnp
from jax import lax
from jax.experimental import pallas as pl
from jax.experimental.pallas import tpu as pltpu
from jax.experimental.pallas import tpu_sc as plsc

B = 16384
V = 100000
D = 128

_NC = 2
_NS = 16
_NW = _NC * _NS
_BPW = B // _NW
_CHUNK = 128
_NCHUNK = _BPW // _CHUNK


def _sc_gather(table, idx):
    mesh = plsc.VectorSubcoreMesh(core_axis_name="c", subcore_axis_name="s")

    @functools.partial(
        pl.kernel,
        out_type=jax.ShapeDtypeStruct((B, D), jnp.float32),
        mesh=mesh,
        scratch_types=[
            pltpu.VMEM((_NCHUNK, _CHUNK), jnp.int32),
            pltpu.VMEM((_BPW, D), jnp.float32),
            pltpu.SemaphoreType.DMA,
        ],
    )
    def gather_kernel(idx_hbm, table_hbm, out_hbm, idx_v, rows_v, sem):
        wid = lax.axis_index("s") * _NC + lax.axis_index("c")
        base = wid * _BPW
        pltpu.sync_copy(idx_hbm.at[wid], idx_v)
        copies = []
        for c in range(_NCHUNK):
            copies.append(
                pltpu.async_copy(
                    table_hbm.at[idx_v.at[c]],
                    rows_v.at[pl.ds(c * _CHUNK, _CHUNK)],
                    sem,
                )
            )
        for cp in copies:
            cp.wait()
        pltpu.sync_copy(rows_v, out_hbm.at[pl.ds(base, _BPW)])

    return gather_kernel(idx.reshape(_NW, _NCHUNK, _CHUNK), table)


def _tc_dense_kernel(rows_ref, coords_ref, nogeo_ref, w_ref, b_ref, nb_ref,
                     out_ref):
    r = rows_ref[...]
    n2 = jnp.sum(r * r, axis=1, keepdims=True)
    inv = lax.rsqrt(n2)
    ng = nogeo_ref[...]
    scale = 1.0 - ng
    cx = coords_ref[:, 0:1] * scale
    cy = coords_ref[:, 1:2] * scale
    w0 = w_ref[0:1, :]
    w1 = w_ref[1:2, :]
    pos = cx * w0 + cy * w1 + b_ref[...] + ng * (nb_ref[...] - b_ref[...])
    res = r * inv + pos
    out_ref[...] = res.T


def _tc_dense(rows, coords, nogeo2d, geo_W, geo_B, nogeo_embed):
    BB = 2048
    grid = B // BB
    return pl.pallas_call(
        _tc_dense_kernel,
        grid=(grid,),
        in_specs=[
            pl.BlockSpec((BB, D), lambda i: (i, 0)),
            pl.BlockSpec((BB, 2), lambda i: (i, 0)),
            pl.BlockSpec((BB, 1), lambda i: (i, 0)),
            pl.BlockSpec((2, D), lambda i: (0, 0)),
            pl.BlockSpec((1, D), lambda i: (0, 0)),
            pl.BlockSpec((1, D), lambda i: (0, 0)),
        ],
        out_specs=pl.BlockSpec((D, BB), lambda i: (0, i)),
        out_shape=jax.ShapeDtypeStruct((D, B), jnp.float32),
    )(rows, coords, nogeo2d, geo_W, geo_B, nogeo_embed)


def kernel(nodes, coords, nogeo, table, geo_W, geo_B, nogeo_embed):
    idx = nodes.astype(jnp.int32)
    rows = _sc_gather(table, idx)
    return _tc_dense(rows, coords, nogeo.reshape(B, 1), geo_W, geo_B,
                     nogeo_embed)


# speedup vs baseline: 2.1930x; 1.0184x over previous
import functools

import jax
import jax.numpy as jnp
from jax import lax
from jax.experimental import pallas as pl
from jax.experimental.pallas import tpu as pltpu
from jax.experimental.pallas import tpu_sc as plsc

B = 16384
V = 100000
D = 128

_NC = 2
_NS = 16
_NW = _NC * _NS
_BPW = B // _NW
_CHUNK = 128
_NCHUNK = _BPW // _CHUNK


def _sc_gather(table, idx):
    mesh = plsc.VectorSubcoreMesh(core_axis_name="c", subcore_axis_name="s")

    @functools.partial(
        pl.kernel,
        out_type=jax.ShapeDtypeStruct((B, D), jnp.float32),
        mesh=mesh,
        scratch_types=[
            pltpu.VMEM((_NCHUNK, _CHUNK), jnp.int32),
            pltpu.VMEM((_BPW, D), jnp.float32),
            pltpu.SemaphoreType.DMA,
        ],
    )
    def gather_kernel(idx_hbm, table_hbm, out_hbm, idx_v, rows_v, sem):
        wid = lax.axis_index("s") * _NC + lax.axis_index("c")
        base = wid * _BPW
        pltpu.sync_copy(idx_hbm.at[wid], idx_v)
        copies = []
        for c in range(_NCHUNK):
            copies.append(
                pltpu.async_copy(
                    table_hbm.at[idx_v.at[c]],
                    rows_v.at[pl.ds(c * _CHUNK, _CHUNK)],
                    sem,
                )
            )
        for cp in copies:
            cp.wait()
        pltpu.sync_copy(rows_v, out_hbm.at[pl.ds(base, _BPW)])

    return gather_kernel(idx.reshape(_NW, _NCHUNK, _CHUNK), table)


def _tc_dense_kernel(rows_ref, coords_ref, nogeo_ref, w_ref, b_ref, nb_ref,
                     out_ref):
    r = rows_ref[...]
    n2 = jnp.sum(r * r, axis=1, keepdims=True)
    inv = lax.rsqrt(n2)
    ng = nogeo_ref[...]
    scale = 1.0 - ng
    cx = coords_ref[:, 0:1] * scale
    cy = coords_ref[:, 1:2] * scale
    w0 = w_ref[0:1, :]
    w1 = w_ref[1:2, :]
    pos = cx * w0 + cy * w1 + b_ref[...] + ng * (nb_ref[...] - b_ref[...])
    res = r * inv + pos
    out_ref[...] = res.T


def _tc_dense(rows, coords, nogeo2d, geo_W, geo_B, nogeo_embed):
    BB = 4096
    grid = B // BB
    return pl.pallas_call(
        _tc_dense_kernel,
        grid=(grid,),
        in_specs=[
            pl.BlockSpec((BB, D), lambda i: (i, 0)),
            pl.BlockSpec((BB, 2), lambda i: (i, 0)),
            pl.BlockSpec((BB, 1), lambda i: (i, 0)),
            pl.BlockSpec((2, D), lambda i: (0, 0)),
            pl.BlockSpec((1, D), lambda i: (0, 0)),
            pl.BlockSpec((1, D), lambda i: (0, 0)),
        ],
        out_specs=pl.BlockSpec((D, BB), lambda i: (0, i)),
        out_shape=jax.ShapeDtypeStruct((D, B), jnp.float32),
    )(rows, coords, nogeo2d, geo_W, geo_B, nogeo_embed)


def kernel(nodes, coords, nogeo, table, geo_W, geo_B, nogeo_embed):
    idx = nodes.astype(jnp.int32)
    rows = _sc_gather(table, idx)
    return _tc_dense(rows, coords, nogeo.reshape(B, 1), geo_W, geo_B,
                     nogeo_embed)
